# baseline (speedup 1.0000x reference)
import functools

import jax
import jax.numpy as jnp
from jax import lax
from jax.experimental import pallas as pl
from jax.experimental.pallas import tpu as pltpu
from jax.experimental.pallas import tpu_sc as plsc

K = 8192
D = 32
B = 2048
ALPHA = 0.01
BIG = 1e10
BT = 256

_NC = 2
_NS = 16
_NW = _NC * _NS
_BPW = B // _NW

_DN = (((1,), (0,)), ((), ()))


def _squash_T(arT, bminT, bmaxT):
    aT = arT / (1.0 + jnp.abs(arT))
    return bminT + (aT + 1.0) * 0.5 * (bmaxT - bminT)


def _body1(x_ref, arT_ref, a1w_ref, a1b_ref, bmin_ref, bmax_ref,
           bminT_ref, bmaxT_ref, mask_ref, khi_ref, klo_ref, xb_ref,
           ptsT_s, p2_s):
    f32 = jnp.float32
    bmin = bmin_ref[:]
    bmax = bmax_ref[:]

    @pl.when(pl.program_id(0) == 0)
    def _():
        ptsT = _squash_T(arT_ref[:], bminT_ref[:], bmaxT_ref[:])
        ptsT_s[:] = ptsT
        p2_s[:] = jnp.sum(ptsT * ptsT, axis=0, keepdims=True)

    xb = x_ref[:]
    xb = xb * jnp.exp(a1w_ref[:]) + a1b_ref[:]
    xb = (jax.nn.sigmoid(xb) - ALPHA) / (1.0 - 2.0 * ALPHA)
    eps = ALPHA / (1.0 - 2.0 * ALPHA)
    xb = (xb + eps) / (1.0 + 2.0 * eps)
    xb = xb * (bmax - bmin) + bmin
    xb_ref[:] = xb

    G = lax.dot_general(xb, ptsT_s[:], _DN, preferred_element_type=f32)
    x2 = jnp.sum(xb * xb, axis=1, keepdims=True)
    d2 = x2 - 2.0 * G + p2_s[:]
    minv = jnp.min(d2, axis=1, keepdims=True)
    kio = lax.broadcasted_iota(jnp.int32, (BT, K), 1)
    k_idx = jnp.min(jnp.where(d2 == minv, kio, K), axis=1, keepdims=True)
    gio = lax.broadcasted_iota(jnp.int32, (BT, K // 32), 1)
    mask_ref[:] = jnp.where(gio == (k_idx >> 5), jnp.int32(1) << (k_idx & 31), 0)
    khi_ref[:] = k_idx >> 2
    klo_ref[:] = k_idx & 3


def _sc_gather(table_hbm, idx_hbm, out_hbm, idx_v, rows_v, sem):
    wid = lax.axis_index("s") * _NC + lax.axis_index("c")
    base = wid * _BPW
    pltpu.sync_copy(idx_hbm.at[pl.ds(base, _BPW)], idx_v)
    pltpu.async_copy(table_hbm.at[idx_v], rows_v, sem).wait()
    pltpu.sync_copy(rows_v, out_hbm.at[pl.ds(base, _BPW)])


def _gather_rows(table, idx):
    mesh = plsc.VectorSubcoreMesh(core_axis_name="c", subcore_axis_name="s")
    f = functools.partial(
        pl.kernel,
        mesh=mesh,
        out_type=jax.ShapeDtypeStruct((B, 4 * D), jnp.float32),
        scratch_types=[
            pltpu.VMEM((_BPW,), jnp.int32),
            pltpu.VMEM((_BPW, 4 * D), jnp.float32),
            pltpu.SemaphoreType.DMA,
        ],
    )(_sc_gather)
    return f(table, idx)


def _body2(xb_ref, zw_ref, klo_ref, arT_ref, scoef_ref, a2w_ref, a2b_ref,
           bmin_ref, bmax_ref, bminT_ref, bmaxT_ref, xout_ref,
           ptsT_s, p2_s):
    f32 = jnp.float32
    bmin = bmin_ref[:]
    bmax = bmax_ref[:]

    @pl.when(pl.program_id(0) == 0)
    def _():
        ptsT = _squash_T(arT_ref[:], bminT_ref[:], bmaxT_ref[:])
        ptsT_s[:] = ptsT
        p2_s[:] = jnp.sum(ptsT * ptsT, axis=0, keepdims=True)

    xb = xb_ref[:]
    zw = zw_ref[:]
    slot = lax.broadcasted_iota(jnp.int32, (BT, 4 * D), 1) // D
    zsel = jnp.where(slot == klo_ref[:], zw, 0.0)
    zraw = (zsel[:, 0:D] + zsel[:, D:2 * D]) + (zsel[:, 2 * D:3 * D] + zsel[:, 3 * D:4 * D])
    az = zraw / (1.0 + jnp.abs(zraw))
    z_k = bmin + (az + 1.0) * 0.5 * (bmax - bmin)

    c = jax.nn.sigmoid(scoef_ref[:]) * 0.98 + 0.01
    xs = xb + c * (z_k - xb)
    dx = xs - z_k

    dxz = jnp.sum(dx * z_k, axis=1, keepdims=True)
    dG = lax.dot_general(dx, ptsT_s[:], _DN, preferred_element_type=f32) - dxz
    zkG = lax.dot_general(z_k, ptsT_s[:], _DN, preferred_element_type=f32)
    zk2 = jnp.sum(z_k * z_k, axis=1, keepdims=True)
    rhs = 0.5 * (p2_s[:] - 2.0 * zkG + zk2)
    den = jnp.where(dG > 1e-12, dG, 1.0)
    t_vor = jnp.min(jnp.where(dG > 1e-12, rhs / den, BIG), axis=1, keepdims=True)
    denb = jnp.where(jnp.abs(dx) > 1e-12, dx, 1.0)
    tb = jnp.where(dx > 1e-12, (bmax - z_k) / denb,
                   jnp.where(dx < -1e-12, (bmin - z_k) / denb, BIG))
    t_box = jnp.min(tb, axis=1, keepdims=True)
    t_max = jnp.minimum(t_vor, t_box)
    r = jnp.sqrt(jnp.sum(dx * dx, axis=1, keepdims=True) + 1e-20)
    s = jnp.clip(r / t_max, 0.0, 1.0 - 1e-5)
    xo = dx / (1.0 - s)
    xout_ref[:] = xo * jnp.exp(a2w_ref[:]) + a2b_ref[:]


def _full(shape):
    return pl.BlockSpec(shape, lambda i: (0,) * len(shape))


def _tile(shape):
    return pl.BlockSpec(shape, lambda i: (i,) + (0,) * (len(shape) - 1))


def kernel(x, anchor_raw, shift_coeff, mixture_logits, an1_w, an1_b, an2_w, an2_b,
           box_min, box_max):
    del mixture_logits
    f32 = jnp.float32
    row = lambda v: v.reshape(1, D)
    col = lambda v: v.reshape(D, 1)
    arT = anchor_raw.T

    maskw, khi, klo, xb = pl.pallas_call(
        _body1,
        grid=(B // BT,),
        in_specs=[
            _tile((BT, D)),
            _full((D, K)),
            _full((1, D)), _full((1, D)),
            _full((1, D)), _full((1, D)),
            _full((D, 1)), _full((D, 1)),
        ],
        out_specs=(_tile((BT, K // 32)), _tile((BT, 1)), _tile((BT, 1)), _tile((BT, D))),
        out_shape=(jax.ShapeDtypeStruct((B, K // 32), jnp.int32),
                   jax.ShapeDtypeStruct((B, 1), jnp.int32),
                   jax.ShapeDtypeStruct((B, 1), jnp.int32),
                   jax.ShapeDtypeStruct((B, D), f32)),
        scratch_shapes=[pltpu.VMEM((D, K), f32), pltpu.VMEM((1, K), f32)],
    )(x, arT, row(an1_w), row(an1_b), row(box_min), row(box_max),
      col(box_min), col(box_max))

    zw = _gather_rows(anchor_raw.reshape(K // 4, 4 * D), khi.reshape(B))

    x_out = pl.pallas_call(
        _body2,
        grid=(B // BT,),
        in_specs=[
            _tile((BT, D)),
            _tile((BT, 4 * D)),
            _tile((BT, 1)),
            _full((D, K)),
            _full((1, D)),
            _full((1, D)), _full((1, D)),
            _full((1, D)), _full((1, D)),
            _full((D, 1)), _full((D, 1)),
        ],
        out_specs=_tile((BT, D)),
        out_shape=jax.ShapeDtypeStruct((B, D), f32),
        scratch_shapes=[pltpu.VMEM((D, K), f32), pltpu.VMEM((1, K), f32)],
    )(xb, zw, klo, arT, row(shift_coeff), row(an2_w), row(an2_b),
      row(box_min), row(box_max), col(box_min), col(box_max))

    shifts = jnp.arange(32, dtype=jnp.int32).reshape(1, 1, 32)
    bits = (maskw.reshape(B, K // 32, 1) >> shifts) & 1
    mask = (bits != 0).reshape(B, 1, K)
    return x_out, mask


# speedup vs baseline: 1.3822x; 1.3822x over previous
import functools

import jax
import jax.numpy as jnp
from jax import lax
from jax.experimental import pallas as pl
from jax.experimental.pallas import tpu as pltpu
from jax.experimental.pallas import tpu_sc as plsc

K = 8192
D = 32
B = 2048
ALPHA = 0.01
BIG = 1e10
BT = 256

_NC = 2
_NS = 16
_NW = _NC * _NS
_BPW = B // _NW

_DN = (((1,), (0,)), ((), ()))


def _squash_T(arT, bminT, bmaxT):
    aT = arT / (1.0 + jnp.abs(arT))
    return bminT + (aT + 1.0) * 0.5 * (bmaxT - bminT)


def _body1(x_ref, arT_ref, a1w_ref, a1b_ref, bmin_ref, bmax_ref,
           bminT_ref, bmaxT_ref, kidx_ref, khi_ref, klo_ref, xb_ref,
           ptsT_s, p2_s):
    f32 = jnp.float32
    bmin = bmin_ref[:]
    bmax = bmax_ref[:]

    @pl.when(pl.program_id(0) == 0)
    def _():
        ptsT = _squash_T(arT_ref[:], bminT_ref[:], bmaxT_ref[:])
        ptsT_s[:] = ptsT
        p2_s[:] = jnp.sum(ptsT * ptsT, axis=0, keepdims=True)

    xb = x_ref[:]
    xb = xb * jnp.exp(a1w_ref[:]) + a1b_ref[:]
    xb = (jax.nn.sigmoid(xb) - ALPHA) / (1.0 - 2.0 * ALPHA)
    eps = ALPHA / (1.0 - 2.0 * ALPHA)
    xb = (xb + eps) / (1.0 + 2.0 * eps)
    xb = xb * (bmax - bmin) + bmin
    xb_ref[:] = xb

    G = lax.dot_general(xb, ptsT_s[:], _DN, preferred_element_type=f32)
    x2 = jnp.sum(xb * xb, axis=1, keepdims=True)
    d2 = x2 - 2.0 * G + p2_s[:]
    minv = jnp.min(d2, axis=1, keepdims=True)
    kio = lax.broadcasted_iota(jnp.int32, (BT, K), 1)
    k_idx = jnp.min(jnp.where(d2 == minv, kio, K), axis=1, keepdims=True)
    kidx_ref[:] = k_idx
    khi_ref[:] = k_idx >> 2
    klo_ref[:] = k_idx & 3


def _sc_gather(table_hbm, idx_hbm, out_hbm, idx_v, rows_v, sem):
    wid = lax.axis_index("s") * _NC + lax.axis_index("c")
    base = wid * _BPW
    pltpu.sync_copy(idx_hbm.at[pl.ds(base, _BPW)], idx_v)
    pltpu.async_copy(table_hbm.at[idx_v], rows_v, sem).wait()
    pltpu.sync_copy(rows_v, out_hbm.at[pl.ds(base, _BPW)])


def _gather_rows(table, idx):
    mesh = plsc.VectorSubcoreMesh(core_axis_name="c", subcore_axis_name="s")
    f = functools.partial(
        pl.kernel,
        mesh=mesh,
        out_type=jax.ShapeDtypeStruct((B, 4 * D), jnp.float32),
        scratch_types=[
            pltpu.VMEM((_BPW,), jnp.int32),
            pltpu.VMEM((_BPW, 4 * D), jnp.float32),
            pltpu.SemaphoreType.DMA,
        ],
    )(_sc_gather)
    return f(table, idx)


def _body2(xb_ref, zw_ref, klo_ref, arT_ref, scoef_ref, a2w_ref, a2b_ref,
           bmin_ref, bmax_ref, bminT_ref, bmaxT_ref, xout_ref,
           ptsT_s, p2_s):
    f32 = jnp.float32
    bmin = bmin_ref[:]
    bmax = bmax_ref[:]

    @pl.when(pl.program_id(0) == 0)
    def _():
        ptsT = _squash_T(arT_ref[:], bminT_ref[:], bmaxT_ref[:])
        ptsT_s[:] = ptsT
        p2_s[:] = jnp.sum(ptsT * ptsT, axis=0, keepdims=True)

    xb = xb_ref[:]
    zw = zw_ref[:]
    slot = lax.broadcasted_iota(jnp.int32, (BT, 4 * D), 1) // D
    zsel = jnp.where(slot == klo_ref[:], zw, 0.0)
    zraw = (zsel[:, 0:D] + zsel[:, D:2 * D]) + (zsel[:, 2 * D:3 * D] + zsel[:, 3 * D:4 * D])
    az = zraw / (1.0 + jnp.abs(zraw))
    z_k = bmin + (az + 1.0) * 0.5 * (bmax - bmin)

    c = jax.nn.sigmoid(scoef_ref[:]) * 0.98 + 0.01
    xs = xb + c * (z_k - xb)
    dx = xs - z_k

    dxz = jnp.sum(dx * z_k, axis=1, keepdims=True)
    dG = lax.dot_general(dx, ptsT_s[:], _DN, preferred_element_type=f32) - dxz
    zkG = lax.dot_general(z_k, ptsT_s[:], _DN, preferred_element_type=f32)
    zk2 = jnp.sum(z_k * z_k, axis=1, keepdims=True)
    rhs = 0.5 * (p2_s[:] - 2.0 * zkG + zk2)
    den = jnp.where(dG > 1e-12, dG, 1.0)
    t_vor = jnp.min(jnp.where(dG > 1e-12, rhs / den, BIG), axis=1, keepdims=True)
    denb = jnp.where(jnp.abs(dx) > 1e-12, dx, 1.0)
    tb = jnp.where(dx > 1e-12, (bmax - z_k) / denb,
                   jnp.where(dx < -1e-12, (bmin - z_k) / denb, BIG))
    t_box = jnp.min(tb, axis=1, keepdims=True)
    t_max = jnp.minimum(t_vor, t_box)
    r = jnp.sqrt(jnp.sum(dx * dx, axis=1, keepdims=True) + 1e-20)
    s = jnp.clip(r / t_max, 0.0, 1.0 - 1e-5)
    xo = dx / (1.0 - s)
    xout_ref[:] = xo * jnp.exp(a2w_ref[:]) + a2b_ref[:]


def _full(shape):
    return pl.BlockSpec(shape, lambda i: (0,) * len(shape))


def _tile(shape):
    return pl.BlockSpec(shape, lambda i: (i,) + (0,) * (len(shape) - 1))


def kernel(x, anchor_raw, shift_coeff, mixture_logits, an1_w, an1_b, an2_w, an2_b,
           box_min, box_max):
    del mixture_logits
    f32 = jnp.float32
    row = lambda v: v.reshape(1, D)
    col = lambda v: v.reshape(D, 1)
    arT = anchor_raw.T

    kidx, khi, klo, xb = pl.pallas_call(
        _body1,
        grid=(B // BT,),
        in_specs=[
            _tile((BT, D)),
            _full((D, K)),
            _full((1, D)), _full((1, D)),
            _full((1, D)), _full((1, D)),
            _full((D, 1)), _full((D, 1)),
        ],
        out_specs=(_tile((BT, 1)), _tile((BT, 1)), _tile((BT, 1)), _tile((BT, D))),
        out_shape=(jax.ShapeDtypeStruct((B, 1), jnp.int32),
                   jax.ShapeDtypeStruct((B, 1), jnp.int32),
                   jax.ShapeDtypeStruct((B, 1), jnp.int32),
                   jax.ShapeDtypeStruct((B, D), f32)),
        scratch_shapes=[pltpu.VMEM((D, K), f32), pltpu.VMEM((1, K), f32)],
    )(x, arT, row(an1_w), row(an1_b), row(box_min), row(box_max),
      col(box_min), col(box_max))

    zw = _gather_rows(anchor_raw.reshape(K // 4, 4 * D), khi.reshape(B))

    x_out = pl.pallas_call(
        _body2,
        grid=(B // BT,),
        in_specs=[
            _tile((BT, D)),
            _tile((BT, 4 * D)),
            _tile((BT, 1)),
            _full((D, K)),
            _full((1, D)),
            _full((1, D)), _full((1, D)),
            _full((1, D)), _full((1, D)),
            _full((D, 1)), _full((D, 1)),
        ],
        out_specs=_tile((BT, D)),
        out_shape=jax.ShapeDtypeStruct((B, D), f32),
        scratch_shapes=[pltpu.VMEM((D, K), f32), pltpu.VMEM((1, K), f32)],
    )(xb, zw, klo, arT, row(shift_coeff), row(an2_w), row(an2_b),
      row(box_min), row(box_max), col(box_min), col(box_max))

    mask = kidx.reshape(B, 1, 1) == jnp.arange(K, dtype=jnp.int32).reshape(1, 1, K)
    return x_out, mask


# speedup vs baseline: 1.4085x; 1.0190x over previous
import functools

import jax
import jax.numpy as jnp
from jax import lax
from jax.experimental import pallas as pl
from jax.experimental.pallas import tpu as pltpu
from jax.experimental.pallas import tpu_sc as plsc

K = 8192
D = 32
B = 2048
ALPHA = 0.01
BIG = 1e10
BT = 512

_NC = 2
_NS = 16
_NW = _NC * _NS
_BPW = B // _NW

_DN = (((1,), (0,)), ((), ()))


def _squash_T(arT, bminT, bmaxT):
    aT = arT / (1.0 + jnp.abs(arT))
    return bminT + (aT + 1.0) * 0.5 * (bmaxT - bminT)


def _body1(x_ref, arT_ref, a1w_ref, a1b_ref, bmin_ref, bmax_ref,
           bminT_ref, bmaxT_ref, kidx_ref, khi_ref, klo_ref, xb_ref,
           ptsT_s, p2_s):
    f32 = jnp.float32
    bmin = bmin_ref[:]
    bmax = bmax_ref[:]

    @pl.when(pl.program_id(0) == 0)
    def _():
        ptsT = _squash_T(arT_ref[:], bminT_ref[:], bmaxT_ref[:])
        ptsT_s[:] = ptsT
        p2_s[:] = jnp.sum(ptsT * ptsT, axis=0, keepdims=True)

    xb = x_ref[:]
    xb = xb * jnp.exp(a1w_ref[:]) + a1b_ref[:]
    xb = (jax.nn.sigmoid(xb) - ALPHA) / (1.0 - 2.0 * ALPHA)
    eps = ALPHA / (1.0 - 2.0 * ALPHA)
    xb = (xb + eps) / (1.0 + 2.0 * eps)
    xb = xb * (bmax - bmin) + bmin
    xb_ref[:] = xb

    G = lax.dot_general(xb, ptsT_s[:], _DN, preferred_element_type=f32)
    x2 = jnp.sum(xb * xb, axis=1, keepdims=True)
    d2 = x2 - 2.0 * G + p2_s[:]
    minv = jnp.min(d2, axis=1, keepdims=True)
    kio = lax.broadcasted_iota(jnp.int32, (BT, K), 1)
    k_idx = jnp.min(jnp.where(d2 == minv, kio, K), axis=1, keepdims=True)
    kidx_ref[:] = k_idx
    khi_ref[:] = k_idx >> 2
    klo_ref[:] = k_idx & 3


def _sc_gather(table_hbm, idx_hbm, out_hbm, idx_v, rows_v, sem):
    wid = lax.axis_index("s") * _NC + lax.axis_index("c")
    base = wid * _BPW
    pltpu.sync_copy(idx_hbm.at[pl.ds(base, _BPW)], idx_v)
    pltpu.async_copy(table_hbm.at[idx_v], rows_v, sem).wait()
    pltpu.sync_copy(rows_v, out_hbm.at[pl.ds(base, _BPW)])


def _gather_rows(table, idx):
    mesh = plsc.VectorSubcoreMesh(core_axis_name="c", subcore_axis_name="s")
    f = functools.partial(
        pl.kernel,
        mesh=mesh,
        out_type=jax.ShapeDtypeStruct((B, 4 * D), jnp.float32),
        scratch_types=[
            pltpu.VMEM((_BPW,), jnp.int32),
            pltpu.VMEM((_BPW, 4 * D), jnp.float32),
            pltpu.SemaphoreType.DMA,
        ],
    )(_sc_gather)
    return f(table, idx)


def _body2(xb_ref, zw_ref, klo_ref, arT_ref, scoef_ref, a2w_ref, a2b_ref,
           bmin_ref, bmax_ref, bminT_ref, bmaxT_ref, xout_ref,
           ptsT_s, p2_s):
    f32 = jnp.float32
    bmin = bmin_ref[:]
    bmax = bmax_ref[:]

    @pl.when(pl.program_id(0) == 0)
    def _():
        ptsT = _squash_T(arT_ref[:], bminT_ref[:], bmaxT_ref[:])
        ptsT_s[:] = ptsT
        p2_s[:] = jnp.sum(ptsT * ptsT, axis=0, keepdims=True)

    xb = xb_ref[:]
    zw = zw_ref[:]
    slot = lax.broadcasted_iota(jnp.int32, (BT, 4 * D), 1) // D
    zsel = jnp.where(slot == klo_ref[:], zw, 0.0)
    zraw = (zsel[:, 0:D] + zsel[:, D:2 * D]) + (zsel[:, 2 * D:3 * D] + zsel[:, 3 * D:4 * D])
    az = zraw / (1.0 + jnp.abs(zraw))
    z_k = bmin + (az + 1.0) * 0.5 * (bmax - bmin)

    c = jax.nn.sigmoid(scoef_ref[:]) * 0.98 + 0.01
    xs = xb + c * (z_k - xb)
    dx = xs - z_k

    dxz = jnp.sum(dx * z_k, axis=1, keepdims=True)
    dG = lax.dot_general(dx, ptsT_s[:], _DN, preferred_element_type=f32) - dxz
    zkG = lax.dot_general(z_k, ptsT_s[:], _DN, preferred_element_type=f32)
    zk2 = jnp.sum(z_k * z_k, axis=1, keepdims=True)
    rhs = 0.5 * (p2_s[:] - 2.0 * zkG + zk2)
    den = jnp.where(dG > 1e-12, dG, 1.0)
    t_vor = jnp.min(jnp.where(dG > 1e-12, rhs / den, BIG), axis=1, keepdims=True)
    denb = jnp.where(jnp.abs(dx) > 1e-12, dx, 1.0)
    tb = jnp.where(dx > 1e-12, (bmax - z_k) / denb,
                   jnp.where(dx < -1e-12, (bmin - z_k) / denb, BIG))
    t_box = jnp.min(tb, axis=1, keepdims=True)
    t_max = jnp.minimum(t_vor, t_box)
    r = jnp.sqrt(jnp.sum(dx * dx, axis=1, keepdims=True) + 1e-20)
    s = jnp.clip(r / t_max, 0.0, 1.0 - 1e-5)
    xo = dx / (1.0 - s)
    xout_ref[:] = xo * jnp.exp(a2w_ref[:]) + a2b_ref[:]


def _full(shape):
    return pl.BlockSpec(shape, lambda i: (0,) * len(shape))


def _tile(shape):
    return pl.BlockSpec(shape, lambda i: (i,) + (0,) * (len(shape) - 1))


def kernel(x, anchor_raw, shift_coeff, mixture_logits, an1_w, an1_b, an2_w, an2_b,
           box_min, box_max):
    del mixture_logits
    f32 = jnp.float32
    row = lambda v: v.reshape(1, D)
    col = lambda v: v.reshape(D, 1)
    arT = anchor_raw.T

    kidx, khi, klo, xb = pl.pallas_call(
        _body1,
        grid=(B // BT,),
        in_specs=[
            _tile((BT, D)),
            _full((D, K)),
            _full((1, D)), _full((1, D)),
            _full((1, D)), _full((1, D)),
            _full((D, 1)), _full((D, 1)),
        ],
        out_specs=(_tile((BT, 1)), _tile((BT, 1)), _tile((BT, 1)), _tile((BT, D))),
        out_shape=(jax.ShapeDtypeStruct((B, 1), jnp.int32),
                   jax.ShapeDtypeStruct((B, 1), jnp.int32),
                   jax.ShapeDtypeStruct((B, 1), jnp.int32),
                   jax.ShapeDtypeStruct((B, D), f32)),
        scratch_shapes=[pltpu.VMEM((D, K), f32), pltpu.VMEM((1, K), f32)],
    )(x, arT, row(an1_w), row(an1_b), row(box_min), row(box_max),
      col(box_min), col(box_max))

    zw = _gather_rows(anchor_raw.reshape(K // 4, 4 * D), khi.reshape(B))

    x_out = pl.pallas_call(
        _body2,
        grid=(B // BT,),
        in_specs=[
            _tile((BT, D)),
            _tile((BT, 4 * D)),
            _tile((BT, 1)),
            _full((D, K)),
            _full((1, D)),
            _full((1, D)), _full((1, D)),
            _full((1, D)), _full((1, D)),
            _full((D, 1)), _full((D, 1)),
        ],
        out_specs=_tile((BT, D)),
        out_shape=jax.ShapeDtypeStruct((B, D), f32),
        scratch_shapes=[pltpu.VMEM((D, K), f32), pltpu.VMEM((1, K), f32)],
    )(xb, zw, klo, arT, row(shift_coeff), row(an2_w), row(an2_b),
      row(box_min), row(box_max), col(box_min), col(box_max))

    mask = kidx.reshape(B, 1, 1) == jnp.arange(K, dtype=jnp.int32).reshape(1, 1, K)
    return x_out, mask


# speedup vs baseline: 1.4461x; 1.0267x over previous
import functools

import jax
import jax.numpy as jnp
from jax import lax
from jax.experimental import pallas as pl
from jax.experimental.pallas import tpu as pltpu
from jax.experimental.pallas import tpu_sc as plsc

K = 8192
D = 32
B = 2048
ALPHA = 0.01
BIG = 1e10
BT = 512

_NC = 2
_NS = 16
_NW = _NC * _NS
_BPW = B // _NW

_DN = (((1,), (0,)), ((), ()))


def _squash_T(arT, bminT, bmaxT):
    aT = arT / (1.0 + jnp.abs(arT))
    return bminT + (aT + 1.0) * 0.5 * (bmaxT - bminT)


def _body1(x_ref, arT_ref, a1w_ref, a1b_ref, bmin_ref, bmax_ref,
           bminT_ref, bmaxT_ref, kidx_ref, xb_ref,
           ptsT_s, p2_s):
    f32 = jnp.float32
    bmin = bmin_ref[:]
    bmax = bmax_ref[:]

    @pl.when(pl.program_id(0) == 0)
    def _():
        ptsT = _squash_T(arT_ref[:], bminT_ref[:], bmaxT_ref[:])
        ptsT_s[:] = ptsT
        p2_s[:] = 0.5 * jnp.sum(ptsT * ptsT, axis=0, keepdims=True)

    xb = x_ref[:]
    xb = xb * jnp.exp(a1w_ref[:]) + a1b_ref[:]
    xb = (jax.nn.sigmoid(xb) - ALPHA) / (1.0 - 2.0 * ALPHA)
    eps = ALPHA / (1.0 - 2.0 * ALPHA)
    xb = (xb + eps) / (1.0 + 2.0 * eps)
    xb = xb * (bmax - bmin) + bmin
    xb_ref[:] = xb

    G = lax.dot_general(xb, ptsT_s[:], _DN, preferred_element_type=f32)
    x2h = 0.5 * jnp.sum(xb * xb, axis=1, keepdims=True)
    d2h = (x2h - G) + p2_s[:]
    minv = jnp.min(d2h, axis=1, keepdims=True)
    kio = lax.broadcasted_iota(jnp.int32, (BT, K), 1)
    k_idx = jnp.min(jnp.where(d2h == minv, kio, K), axis=1, keepdims=True)
    kidx_ref[:] = k_idx


def _sc_gather(table_hbm, idx_hbm, out_hbm, idx_v, rows_v, sem):
    wid = lax.axis_index("s") * _NC + lax.axis_index("c")
    base = wid * _BPW
    pltpu.sync_copy(idx_hbm.at[pl.ds(base, _BPW)], idx_v)
    pltpu.async_copy(table_hbm.at[idx_v], rows_v, sem).wait()
    pltpu.sync_copy(rows_v, out_hbm.at[pl.ds(base, _BPW)])


def _gather_rows(table, idx):
    mesh = plsc.VectorSubcoreMesh(core_axis_name="c", subcore_axis_name="s")
    f = functools.partial(
        pl.kernel,
        mesh=mesh,
        out_type=jax.ShapeDtypeStruct((B, 4 * D), jnp.float32),
        scratch_types=[
            pltpu.VMEM((_BPW,), jnp.int32),
            pltpu.VMEM((_BPW, 4 * D), jnp.float32),
            pltpu.SemaphoreType.DMA,
        ],
    )(_sc_gather)
    return f(table, idx)


def _body2(xb_ref, zw_ref, klo_ref, arT_ref, scoef_ref, a2w_ref, a2b_ref,
           bmin_ref, bmax_ref, bminT_ref, bmaxT_ref, xout_ref,
           ptsT_s, p2_s):
    f32 = jnp.float32
    bmin = bmin_ref[:]
    bmax = bmax_ref[:]

    @pl.when(pl.program_id(0) == 0)
    def _():
        ptsT = _squash_T(arT_ref[:], bminT_ref[:], bmaxT_ref[:])
        ptsT_s[:] = ptsT
        p2_s[:] = 0.5 * jnp.sum(ptsT * ptsT, axis=0, keepdims=True)

    xb = xb_ref[:]
    zw = zw_ref[:]
    slot = lax.broadcasted_iota(jnp.int32, (BT, 4 * D), 1) // D
    zsel = jnp.where(slot == klo_ref[:], zw, 0.0)
    zraw = (zsel[:, 0:D] + zsel[:, D:2 * D]) + (zsel[:, 2 * D:3 * D] + zsel[:, 3 * D:4 * D])
    az = zraw / (1.0 + jnp.abs(zraw))
    z_k = bmin + (az + 1.0) * 0.5 * (bmax - bmin)

    c = jax.nn.sigmoid(scoef_ref[:]) * 0.98 + 0.01
    xs = xb + c * (z_k - xb)
    dx = xs - z_k

    dxz = jnp.sum(dx * z_k, axis=1, keepdims=True)
    dG = lax.dot_general(dx, ptsT_s[:], _DN, preferred_element_type=f32) - dxz
    zkG = lax.dot_general(z_k, ptsT_s[:], _DN, preferred_element_type=f32)
    zk2h = 0.5 * jnp.sum(z_k * z_k, axis=1, keepdims=True)
    rhs = (p2_s[:] - zkG) + zk2h
    den = jnp.where(dG > 1e-12, dG, 1.0)
    t_vor = jnp.min(jnp.where(dG > 1e-12, rhs / den, BIG), axis=1, keepdims=True)
    denb = jnp.where(jnp.abs(dx) > 1e-12, dx, 1.0)
    tb = jnp.where(dx > 1e-12, (bmax - z_k) / denb,
                   jnp.where(dx < -1e-12, (bmin - z_k) / denb, BIG))
    t_box = jnp.min(tb, axis=1, keepdims=True)
    t_max = jnp.minimum(t_vor, t_box)
    r = jnp.sqrt(jnp.sum(dx * dx, axis=1, keepdims=True) + 1e-20)
    s = jnp.clip(r / t_max, 0.0, 1.0 - 1e-5)
    xo = dx / (1.0 - s)
    xout_ref[:] = xo * jnp.exp(a2w_ref[:]) + a2b_ref[:]


def _full(shape):
    return pl.BlockSpec(shape, lambda i: (0,) * len(shape))


def _tile(shape):
    return pl.BlockSpec(shape, lambda i: (i,) + (0,) * (len(shape) - 1))


def kernel(x, anchor_raw, shift_coeff, mixture_logits, an1_w, an1_b, an2_w, an2_b,
           box_min, box_max):
    del mixture_logits
    f32 = jnp.float32
    row = lambda v: v.reshape(1, D)
    col = lambda v: v.reshape(D, 1)
    arT = anchor_raw.T

    kidx, xb = pl.pallas_call(
        _body1,
        grid=(B // BT,),
        in_specs=[
            _tile((BT, D)),
            _full((D, K)),
            _full((1, D)), _full((1, D)),
            _full((1, D)), _full((1, D)),
            _full((D, 1)), _full((D, 1)),
        ],
        out_specs=(_tile((BT, 1)), _tile((BT, D))),
        out_shape=(jax.ShapeDtypeStruct((B, 1), jnp.int32),
                   jax.ShapeDtypeStruct((B, D), f32)),
        scratch_shapes=[pltpu.VMEM((D, K), f32), pltpu.VMEM((1, K), f32)],
    )(x, arT, row(an1_w), row(an1_b), row(box_min), row(box_max),
      col(box_min), col(box_max))

    khi = (kidx >> 2).reshape(B)
    klo = kidx & 3
    zw = _gather_rows(anchor_raw.reshape(K // 4, 4 * D), khi)

    x_out = pl.pallas_call(
        _body2,
        grid=(B // BT,),
        in_specs=[
            _tile((BT, D)),
            _tile((BT, 4 * D)),
            _tile((BT, 1)),
            _full((D, K)),
            _full((1, D)),
            _full((1, D)), _full((1, D)),
            _full((1, D)), _full((1, D)),
            _full((D, 1)), _full((D, 1)),
        ],
        out_specs=_tile((BT, D)),
        out_shape=jax.ShapeDtypeStruct((B, D), f32),
        scratch_shapes=[pltpu.VMEM((D, K), f32), pltpu.VMEM((1, K), f32)],
    )(xb, zw, klo, arT, row(shift_coeff), row(an2_w), row(an2_b),
      row(box_min), row(box_max), col(box_min), col(box_max))

    mask = kidx.reshape(B, 1, 1) == jnp.arange(K, dtype=jnp.int32).reshape(1, 1, K)
    return x_out, mask
